# Initial kernel scaffold; baseline (speedup 1.0000x reference)
import jax
import jax.numpy as jnp
from jax.experimental import pallas as pl


def kernel(x, segment_label, token_table, segment_table):
    raise NotImplementedError("write your pallas kernel here")


# speedup vs baseline: 1.2713x; 1.2713x over previous
import math

import jax
import jax.numpy as jnp
from jax import lax
from jax.experimental import pallas as pl
from jax.experimental.pallas import tpu as pltpu
from jax.experimental.pallas import tpu_sc as plsc

D = 64
B_TOTAL = 1024 * 200
NC, NS, L = 2, 16, 16
NW = NC * NS
ROWS_W = B_TOTAL // NW
GROUP = 128
NGROUP = ROWS_W // GROUP
NBUF = 2
SEQ = 200
NSEG = 3


def _sinusoidal_pe(seq_len, d_model):
    pos = jnp.arange(seq_len, dtype=jnp.float32)[:, None]
    div = jnp.exp(
        jnp.arange(0, d_model, 2, dtype=jnp.float32)
        * (-math.log(10000.0) / d_model)
    )
    pe = jnp.zeros((seq_len, d_model), dtype=jnp.float32)
    pe = pe.at[:, 0::2].set(jnp.sin(pos * div))
    pe = pe.at[:, 1::2].set(jnp.cos(pos * div))
    return pe


def _body(x_hbm, seg_hbm, tok_hbm, comb_hbm, out_hbm,
          idx_v, cidx_v, tok_bufs, comb_bufs, out_bufs, gsems, ssems):
    wid = lax.axis_index("s") * NC + lax.axis_index("c")
    base = wid * ROWS_W

    pltpu.sync_copy(x_hbm.at[pl.ds(base, ROWS_W)], idx_v)
    pltpu.sync_copy(seg_hbm.at[pl.ds(base, ROWS_W)], cidx_v)

    iota = lax.iota(jnp.int32, L)
    def cidx_body(j, carry):
        off = j * L
        posv = (off + iota) % SEQ
        segv = cidx_v[pl.ds(off, L)]
        cidx_v[pl.ds(off, L)] = segv * SEQ + posv
        return carry
    lax.fori_loop(0, ROWS_W // L, cidx_body, 0)

    def gathers_start(g, b):
        pltpu.make_async_copy(
            tok_hbm.at[idx_v.at[pl.ds(g * GROUP, GROUP)]],
            tok_bufs[b], gsems[b]).start()
        pltpu.make_async_copy(
            comb_hbm.at[cidx_v.at[pl.ds(g * GROUP, GROUP)]],
            comb_bufs[b], gsems[b]).start()

    def gathers_wait(g, b):
        pltpu.make_async_copy(
            tok_hbm.at[idx_v.at[pl.ds(g * GROUP, GROUP)]],
            tok_bufs[b], gsems[b]).wait()
        pltpu.make_async_copy(
            comb_hbm.at[cidx_v.at[pl.ds(g * GROUP, GROUP)]],
            comb_bufs[b], gsems[b]).wait()

    def scatter_start(g, b):
        pltpu.make_async_copy(
            out_bufs[b], out_hbm.at[pl.ds(base + g * GROUP, GROUP)],
            ssems[b]).start()

    def scatter_wait(g, b):
        pltpu.make_async_copy(
            out_bufs[b], out_hbm.at[pl.ds(base + g * GROUP, GROUP)],
            ssems[b]).wait()

    for b in range(NBUF):
        gathers_start(b, b)

    def outer(og, carry):
        for b in range(NBUF):
            g = og * NBUF + b
            gathers_wait(g, b)
            @pl.when(og > 0)
            def _():
                scatter_wait(g - NBUF, b)

            def row_body(t, rcarry):
                for c in range(D // L):
                    tv = tok_bufs[b][t, pl.ds(c * L, L)]
                    cv = comb_bufs[b][t, pl.ds(c * L, L)]
                    out_bufs[b][t, pl.ds(c * L, L)] = tv * 8.0 + cv
                return rcarry
            lax.fori_loop(0, GROUP, row_body, 0)

            @pl.when(g + NBUF < NGROUP)
            def _():
                gathers_start(g + NBUF, b)
            scatter_start(g, b)
        return carry
    lax.fori_loop(0, NGROUP // NBUF, outer, 0)

    for b in range(NBUF):
        scatter_wait(NGROUP - NBUF + b, b)


def _sc_embed(x_flat, seg_flat, token_table, comb):
    mesh = plsc.VectorSubcoreMesh(core_axis_name="c", subcore_axis_name="s")

    def body(x_hbm, seg_hbm, tok_hbm, comb_hbm, out_hbm,
             idx_v, cidx_v, tb0, tb1, cb0, cb1, ob0, ob1,
             gs0, gs1, ss0, ss1):
        _body(x_hbm, seg_hbm, tok_hbm, comb_hbm, out_hbm,
              idx_v, cidx_v, (tb0, tb1), (cb0, cb1), (ob0, ob1),
              (gs0, gs1), (ss0, ss1))

    run = pl.kernel(
        body,
        out_type=jax.ShapeDtypeStruct((B_TOTAL, D), jnp.float32),
        mesh=mesh,
        scratch_types=[
            pltpu.VMEM((ROWS_W,), jnp.int32),
            pltpu.VMEM((ROWS_W,), jnp.int32),
            pltpu.VMEM((GROUP, D), jnp.float32),
            pltpu.VMEM((GROUP, D), jnp.float32),
            pltpu.VMEM((GROUP, D), jnp.float32),
            pltpu.VMEM((GROUP, D), jnp.float32),
            pltpu.VMEM((GROUP, D), jnp.float32),
            pltpu.VMEM((GROUP, D), jnp.float32),
            pltpu.SemaphoreType.DMA,
            pltpu.SemaphoreType.DMA,
            pltpu.SemaphoreType.DMA,
            pltpu.SemaphoreType.DMA,
        ],
        compiler_params=pltpu.CompilerParams(use_tc_tiling_on_sc=False),
    )
    return run(x_flat, seg_flat, token_table, comb)


def kernel(x, segment_label, token_table, segment_table):
    batch, seq = x.shape
    x_flat = x.reshape(-1).astype(jnp.int32)
    seg_flat = segment_label.reshape(-1).astype(jnp.int32)
    pe = _sinusoidal_pe(seq, D)
    comb = (segment_table[:, None, :] + pe[None, :, :]).reshape(NSEG * SEQ, D)
    out = _sc_embed(x_flat, seg_flat, token_table, comb)
    return out.reshape(batch, seq, D)
